# baseline (speedup 1.0000x reference)
import jax
import jax.numpy as jnp
from jax import lax
from jax.experimental import pallas as pl
from jax.experimental.pallas import tpu as pltpu
from jax.experimental.pallas import tpu_sc as plsc

N = 50000
D = 128
E = 200000
R = 3

NSC = 2
NSUB = 16
NCHUNK = 6
C = 8448
NP = NCHUNK * C
EW = 12544
EP = NSUB * EW
SLEN = 1568
NSTRIP = EW // SLEN
NGS = SLEN // 16
B = 128
CAP = EW + B
RPW = C // NSUB


def _sc_body(x_hbm, src_hbm, dst_hbm, agg_hbm,
             strip_src, strip_dst, comp, srcidx_v, dstidx_v,
             rows_v, deg_local, deg_acc, deg_tmp, agg_sh, deg_all, sem):
  c = lax.axis_index("c")
  s = lax.axis_index("s")
  zf16 = jnp.zeros((16,), jnp.float32)
  one16 = jnp.ones((16,), jnp.float32)
  dump16 = jnp.full((16,), C << 16, jnp.int32)

  def zero_rows_v(i, carry):
    for j in range(D // 16):
      rows_v[i, pl.ds(j * 16, 16)] = zf16
    return carry

  def zero_vec(ref, n16):
    def z(i, carry):
      ref[pl.ds(i * 16, 16)] = zf16
      return carry
    lax.fori_loop(0, n16, z, 0)

  lax.fori_loop(0, B, zero_rows_v, 0)

  base = s * RPW
  nfull = RPW // B
  rem = RPW - nfull * B

  def zero_agg():
    for t in range(nfull):
      pltpu.sync_copy(rows_v, agg_sh.at[pl.ds(base + t * B, B)])
    pltpu.sync_copy(rows_v.at[pl.ds(0, rem)],
                    agg_sh.at[pl.ds(base + nfull * B, rem)])

  zero_agg()
  plsc.subcore_barrier()

  for r in range(R):
    for k in range(NCHUNK // NSC):
      chunk = c * (NCHUNK // NSC) + k
      lo = chunk * C

      zero_vec(deg_local, C // 16)
      cur = jnp.int32(0)
      for t in range(NSTRIP):
        off = r * EP + s * EW + t * SLEN
        pltpu.sync_copy(src_hbm.at[pl.ds(off, SLEN)], strip_src)
        pltpu.sync_copy(dst_hbm.at[pl.ds(off, SLEN)], strip_dst)

        def compact(g, cur):
          sv = strip_src[pl.ds(g * 16, 16)]
          dv = strip_dst[pl.ds(g * 16, 16)]
          m = (dv >= lo) & (dv < lo + C)
          dvl = jnp.where(m, dv - lo, 0)
          packed = jnp.where(m, sv + (dvl << 16), C << 16)
          key = jnp.where(m, 0, 1)
          _, pk = plsc.sort_key_val(key, packed)
          cntv = plsc.all_reduce_population_count(m)
          comp[pl.ds(cur, 16)] = pk
          plsc.addupdate_scatter(deg_local, [dvl], one16, mask=m)
          return cur + cntv[0]

        cur = lax.fori_loop(0, NGS, compact, cur)

      for j in range(B // 16):
        comp[pl.ds(cur + j * 16, 16)] = dump16

      nbat = (cur + (B - 1)) // B

      def batch(i, carry):
        for j in range(B // 16):
          pk = comp[pl.ds(i * B + j * 16, 16)]
          srcidx_v[pl.ds(j * 16, 16)] = pk & 0xFFFF
          dstidx_v[pl.ds(j * 16, 16)] = pk >> 16
        pltpu.async_copy(x_hbm.at[srcidx_v], rows_v, sem).wait()
        pltpu.sync_copy(rows_v, agg_sh.at[dstidx_v], add=True)
        return carry

      lax.fori_loop(0, nbat, batch, 0)

      pltpu.sync_copy(deg_local, deg_all.at[pl.ds(s * C, C)])
      plsc.subcore_barrier()

      zero_vec(deg_acc, RPW // 16)
      for t in range(NSUB):
        pltpu.sync_copy(deg_all.at[pl.ds(t * C + base, RPW)], deg_tmp)

        def addj(j, carry):
          deg_acc[pl.ds(j * 16, 16)] = (deg_acc[pl.ds(j * 16, 16)]
                                        + deg_tmp[pl.ds(j * 16, 16)])
          return carry

        lax.fori_loop(0, RPW // 16, addj, 0)

      def invj(j, carry):
        dv16 = deg_acc[pl.ds(j * 16, 16)]
        deg_acc[pl.ds(j * 16, 16)] = 1.0 / jnp.maximum(dv16, 1.0)
        return carry

      lax.fori_loop(0, RPW // 16, invj, 0)

      DG = 48

      def dump_grp(g, carry):
        pltpu.sync_copy(agg_sh.at[pl.ds(base + g * DG, DG)],
                        rows_v.at[pl.ds(0, DG)])

        def rowi(i, carry2):
          scale = plsc.load_gather(
              deg_acc, [jnp.full((16,), g * DG + i, jnp.int32)])
          for j in range(D // 16):
            rows_v[i, pl.ds(j * 16, 16)] = (rows_v[i, pl.ds(j * 16, 16)]
                                            * scale)
          return carry2

        lax.fori_loop(0, DG, rowi, 0)
        pltpu.sync_copy(rows_v.at[pl.ds(0, DG)],
                        agg_hbm.at[r, pl.ds(lo + base + g * DG, DG)])
        return carry

      lax.fori_loop(0, RPW // DG, dump_grp, 0)

      lax.fori_loop(0, B, zero_rows_v, 0)
      zero_agg()
      plsc.subcore_barrier()


def _make_sc_kernel():
  return pl.kernel(
      _sc_body,
      out_type=[
          jax.ShapeDtypeStruct((R, NP, D), jnp.float32),
      ],
      mesh=plsc.VectorSubcoreMesh(
          core_axis_name="c", subcore_axis_name="s",
          num_cores=NSC, num_subcores=NSUB),
      compiler_params=pltpu.CompilerParams(needs_layout_passes=False),
      scratch_types=[
          pltpu.VMEM((SLEN,), jnp.int32),
          pltpu.VMEM((SLEN,), jnp.int32),
          pltpu.VMEM((CAP,), jnp.int32),
          pltpu.VMEM((B,), jnp.int32),
          pltpu.VMEM((B,), jnp.int32),
          pltpu.VMEM((B, D), jnp.float32),
          pltpu.VMEM((C,), jnp.float32),
          pltpu.VMEM((RPW,), jnp.float32),
          pltpu.VMEM((RPW,), jnp.float32),
          pltpu.VMEM_SHARED((C + 8, D), jnp.float32),
          pltpu.VMEM_SHARED((NSUB * C,), jnp.float32),
          pltpu.SemaphoreType.DMA,
      ],
  )


BN = 400


def _tc_body(agg_ref, x_ref, w_ref, lw_ref, out_ref):
  acc = jnp.dot(x_ref[...], lw_ref[...], preferred_element_type=jnp.float32)
  for r in range(R):
    acc = acc + jnp.dot(agg_ref[r], w_ref[r],
                        preferred_element_type=jnp.float32)
  out_ref[...] = acc


def _tc_epilogue(agg, x, weight, loop_weight):
  return pl.pallas_call(
      _tc_body,
      grid=(N // BN,),
      in_specs=[
          pl.BlockSpec((R, BN, D), lambda i: (0, i, 0)),
          pl.BlockSpec((BN, D), lambda i: (i, 0)),
          pl.BlockSpec((R, D, D), lambda i: (0, 0, 0)),
          pl.BlockSpec((D, D), lambda i: (0, 0)),
      ],
      out_specs=pl.BlockSpec((BN, D), lambda i: (i, 0)),
      out_shape=jax.ShapeDtypeStruct((N, D), jnp.float32),
  )(agg, x, weight, loop_weight)


@jax.jit
def kernel(x, edge_index_r0, edge_index_r1, edge_index_r2, weight, loop_weight):
  srcs = jnp.stack([edge_index_r0[0], edge_index_r1[0], edge_index_r2[0]])
  dsts = jnp.stack([edge_index_r0[1], edge_index_r1[1], edge_index_r2[1]])
  srcs = jnp.pad(srcs, ((0, 0), (0, EP - E))).reshape(-1)
  dsts = jnp.pad(dsts, ((0, 0), (0, EP - E)), constant_values=NP).reshape(-1)
  (agg,) = _make_sc_kernel()(x, srcs, dsts)
  return _tc_epilogue(agg, x, weight, loop_weight)


# speedup vs baseline: 1.2167x; 1.2167x over previous
import jax
import jax.numpy as jnp
from jax import lax
from jax.experimental import pallas as pl
from jax.experimental.pallas import tpu as pltpu
from jax.experimental.pallas import tpu_sc as plsc

N = 50000
D = 128
E = 200000
R = 3

NSC = 2
NSUB = 16
NCHUNK = 6
C = 8448
NP = NCHUNK * C
EW = 12544
EP = NSUB * EW
SLEN = 1568
NSTRIP = EW // SLEN
NGS = SLEN // 16
B = 32
CAP = EW + B
RPW = C // NSUB


def _sc_body(x_hbm, src_hbm, dst_hbm, agg_hbm,
             strip_src, strip_dst, comp, srcidx_v, dstidx_v,
             rows_v, deg_local, deg_acc, deg_tmp, agg_sh, deg_all, sem):
  c = lax.axis_index("c")
  s = lax.axis_index("s")
  zf16 = jnp.zeros((16,), jnp.float32)
  one16 = jnp.ones((16,), jnp.float32)
  dump16 = jnp.full((16,), C << 16, jnp.int32)

  def zero_rows_v(i, carry):
    for j in range(D // 16):
      rows_v[i, pl.ds(j * 16, 16)] = zf16
    return carry

  def zero_vec(ref, n16):
    def z(i, carry):
      ref[pl.ds(i * 16, 16)] = zf16
      return carry
    lax.fori_loop(0, n16, z, 0)

  lax.fori_loop(0, B, zero_rows_v, 0)

  base = s * RPW
  nfull = RPW // B
  rem = RPW - nfull * B

  def zero_agg():
    for t in range(nfull):
      pltpu.sync_copy(rows_v, agg_sh.at[pl.ds(base + t * B, B)])
    pltpu.sync_copy(rows_v.at[pl.ds(0, rem)],
                    agg_sh.at[pl.ds(base + nfull * B, rem)])

  zero_agg()
  plsc.subcore_barrier()

  for r in range(R):
    for k in range(NCHUNK // NSC):
      chunk = c * (NCHUNK // NSC) + k
      lo = chunk * C

      zero_vec(deg_local, C // 16)
      cur = jnp.int32(0)
      for t in range(NSTRIP):
        off = r * EP + s * EW + t * SLEN
        pltpu.sync_copy(src_hbm.at[pl.ds(off, SLEN)], strip_src)
        pltpu.sync_copy(dst_hbm.at[pl.ds(off, SLEN)], strip_dst)

        def compact(g, cur):
          sv = strip_src[pl.ds(g * 16, 16)]
          dv = strip_dst[pl.ds(g * 16, 16)]
          m = (dv >= lo) & (dv < lo + C)
          dvl = jnp.where(m, dv - lo, 0)
          packed = jnp.where(m, sv + (dvl << 16), C << 16)
          key = jnp.where(m, 0, 1)
          _, pk = plsc.sort_key_val(key, packed)
          cntv = plsc.all_reduce_population_count(m)
          comp[pl.ds(cur, 16)] = pk
          plsc.addupdate_scatter(deg_local, [dvl], one16, mask=m)
          return cur + cntv[0]

        cur = lax.fori_loop(0, NGS, compact, cur)

      for j in range(B // 16):
        comp[pl.ds(cur + j * 16, 16)] = dump16

      nbat = (cur + (B - 1)) // B

      def batch(i, carry):
        for j in range(B // 16):
          pk = comp[pl.ds(i * B + j * 16, 16)]
          srcidx_v[pl.ds(j * 16, 16)] = pk & 0xFFFF
          dstidx_v[pl.ds(j * 16, 16)] = pk >> 16
        pltpu.async_copy(x_hbm.at[srcidx_v], rows_v, sem).wait()
        pltpu.sync_copy(rows_v, agg_sh.at[dstidx_v], add=True)
        return carry

      lax.fori_loop(0, nbat, batch, 0)

      pltpu.sync_copy(deg_local, deg_all.at[pl.ds(s * C, C)])
      plsc.subcore_barrier()

      zero_vec(deg_acc, RPW // 16)
      for t in range(NSUB):
        pltpu.sync_copy(deg_all.at[pl.ds(t * C + base, RPW)], deg_tmp)

        def addj(j, carry):
          deg_acc[pl.ds(j * 16, 16)] = (deg_acc[pl.ds(j * 16, 16)]
                                        + deg_tmp[pl.ds(j * 16, 16)])
          return carry

        lax.fori_loop(0, RPW // 16, addj, 0)

      def invj(j, carry):
        dv16 = deg_acc[pl.ds(j * 16, 16)]
        deg_acc[pl.ds(j * 16, 16)] = 1.0 / jnp.maximum(dv16, 1.0)
        return carry

      lax.fori_loop(0, RPW // 16, invj, 0)

      DG = 48

      def dump_grp(g, carry):
        pltpu.sync_copy(agg_sh.at[pl.ds(base + g * DG, DG)],
                        rows_v.at[pl.ds(0, DG)])

        def rowi(i, carry2):
          scale = plsc.load_gather(
              deg_acc, [jnp.full((16,), g * DG + i, jnp.int32)])
          for j in range(D // 16):
            rows_v[i, pl.ds(j * 16, 16)] = (rows_v[i, pl.ds(j * 16, 16)]
                                            * scale)
          return carry2

        lax.fori_loop(0, DG, rowi, 0)
        pltpu.sync_copy(rows_v.at[pl.ds(0, DG)],
                        agg_hbm.at[r, pl.ds(lo + base + g * DG, DG)])
        return carry

      lax.fori_loop(0, RPW // DG, dump_grp, 0)

      lax.fori_loop(0, B, zero_rows_v, 0)
      zero_agg()
      plsc.subcore_barrier()


def _make_sc_kernel():
  return pl.kernel(
      _sc_body,
      out_type=[
          jax.ShapeDtypeStruct((R, NP, D), jnp.float32),
      ],
      mesh=plsc.VectorSubcoreMesh(
          core_axis_name="c", subcore_axis_name="s",
          num_cores=NSC, num_subcores=NSUB),
      compiler_params=pltpu.CompilerParams(needs_layout_passes=False),
      scratch_types=[
          pltpu.VMEM((SLEN,), jnp.int32),
          pltpu.VMEM((SLEN,), jnp.int32),
          pltpu.VMEM((CAP,), jnp.int32),
          pltpu.VMEM((B,), jnp.int32),
          pltpu.VMEM((B,), jnp.int32),
          pltpu.VMEM((B, D), jnp.float32),
          pltpu.VMEM((C,), jnp.float32),
          pltpu.VMEM((RPW,), jnp.float32),
          pltpu.VMEM((RPW,), jnp.float32),
          pltpu.VMEM_SHARED((C + 8, D), jnp.float32),
          pltpu.VMEM_SHARED((NSUB * C,), jnp.float32),
          pltpu.SemaphoreType.DMA,
      ],
  )


BN = 400


def _tc_body(agg_ref, x_ref, w_ref, lw_ref, out_ref):
  acc = jnp.dot(x_ref[...], lw_ref[...], preferred_element_type=jnp.float32)
  for r in range(R):
    acc = acc + jnp.dot(agg_ref[r], w_ref[r],
                        preferred_element_type=jnp.float32)
  out_ref[...] = acc


def _tc_epilogue(agg, x, weight, loop_weight):
  return pl.pallas_call(
      _tc_body,
      grid=(N // BN,),
      in_specs=[
          pl.BlockSpec((R, BN, D), lambda i: (0, i, 0)),
          pl.BlockSpec((BN, D), lambda i: (i, 0)),
          pl.BlockSpec((R, D, D), lambda i: (0, 0, 0)),
          pl.BlockSpec((D, D), lambda i: (0, 0)),
      ],
      out_specs=pl.BlockSpec((BN, D), lambda i: (i, 0)),
      out_shape=jax.ShapeDtypeStruct((N, D), jnp.float32),
  )(agg, x, weight, loop_weight)


@jax.jit
def kernel(x, edge_index_r0, edge_index_r1, edge_index_r2, weight, loop_weight):
  srcs = jnp.stack([edge_index_r0[0], edge_index_r1[0], edge_index_r2[0]])
  dsts = jnp.stack([edge_index_r0[1], edge_index_r1[1], edge_index_r2[1]])
  srcs = jnp.pad(srcs, ((0, 0), (0, EP - E))).reshape(-1)
  dsts = jnp.pad(dsts, ((0, 0), (0, EP - E)), constant_values=NP).reshape(-1)
  (agg,) = _make_sc_kernel()(x, srcs, dsts)
  return _tc_epilogue(agg, x, weight, loop_weight)
